# baseline (speedup 1.0000x reference)
import functools
import math

import jax
import jax.numpy as jnp
from jax import lax
from jax.experimental import pallas as pl
from jax.experimental.pallas import tpu as pltpu
from jax.experimental.pallas import tpu_sc as plsc

L = 2048
H = 16
E = 128
NNZ = 32768
NUM_SC = 2
NUM_TILES = 16
WIN_ROWS = 512
WINSZ = WIN_ROWS * L
PASSES = (L // NUM_SC) // WIN_ROWS
EDGES_PER_TILE = NNZ // NUM_TILES
TILE_SLICE = WINSZ // NUM_TILES
ZCHUNK = 8192
PAD = 256


def _build_counts(src, dst):
    mesh = plsc.VectorSubcoreMesh(core_axis_name="c", subcore_axis_name="s")

    @functools.partial(
        pl.kernel,
        out_type=jax.ShapeDtypeStruct((L, L), jnp.float32),
        mesh=mesh,
        scratch_types=[
            pltpu.VMEM((EDGES_PER_TILE,), jnp.int32),
            pltpu.VMEM((EDGES_PER_TILE,), jnp.int32),
            pltpu.VMEM((16, 128), jnp.int32),
            pltpu.VMEM((128,), jnp.float32),
            pltpu.VMEM((ZCHUNK,), jnp.float32),
            pltpu.VMEM((2, 8, L), jnp.float32),
            pltpu.VMEM_SHARED((WINSZ + PAD,), jnp.float32),
            pltpu.SemaphoreType.DMA,
            pltpu.SemaphoreType.DMA,
        ],
    )
    def builder(src_hbm, dst_hbm, c_hbm, src_v, dst_v, idx_b, ones_v, zbuf,
                bbuf, window, sem, wsem):
        cid = lax.axis_index("c")
        sid = lax.axis_index("s")

        h_src = pltpu.async_copy(
            src_hbm.at[pl.ds(sid * EDGES_PER_TILE, EDGES_PER_TILE)], src_v, sem)
        h_dst = pltpu.async_copy(
            dst_hbm.at[pl.ds(sid * EDGES_PER_TILE, EDGES_PER_TILE)], dst_v, sem)

        def fill_zeros(i, _):
            zbuf[pl.ds(i * 16, 16)] = jnp.zeros((16,), jnp.float32)
            return 0
        lax.fori_loop(0, ZCHUNK // 16, fill_zeros, 0)
        def fill_ones(i, _):
            ones_v[pl.ds(i * 16, 16)] = jnp.ones((16,), jnp.float32)
            return 0
        lax.fori_loop(0, 128 // 16, fill_ones, 0)
        h_src.wait()
        h_dst.wait()

        lane = lax.iota(jnp.int32, 16)
        dump_base = WINSZ + sid * 16

        for p in range(PASSES):
            row_base = cid * (PASSES * WIN_ROWS) + p * WIN_ROWS

            zh = [
                pltpu.async_copy(
                    zbuf, window.at[pl.ds(sid * TILE_SLICE + i * ZCHUNK, ZCHUNK)],
                    sem)
                for i in range(TILE_SLICE // ZCHUNK)
            ]
            for j in range(EDGES_PER_TILE // 128):
                for g in range(8):
                    off = j * 128 + g * 16
                    s16 = src_v[pl.ds(off, 16)]
                    d16 = dst_v[pl.ds(off, 16)]
                    r = s16 - row_base
                    in_win = (r >= 0) & (r < WIN_ROWS)
                    idx_b[j, pl.ds(g * 16, 16)] = jnp.where(
                        in_win, r * L + d16, dump_base + lane)
            for h in zh:
                h.wait()
            plsc.subcore_barrier()

            sh = [
                pltpu.async_copy(ones_v, window.at[idx_b.at[j]], sem, add=True)
                for j in range(EDGES_PER_TILE // 128)
            ]
            for h in sh:
                h.wait()
            plsc.subcore_barrier()

            wh = []
            for i in range(TILE_SLICE // (8 * L)):
                buf = bbuf.at[i % 2]
                if i >= 2:
                    wh[i - 2].wait()
                rh = [
                    pltpu.async_copy(
                        window.at[pl.ds(sid * TILE_SLICE + i * (8 * L) + r * L, L)],
                        buf.at[r], sem)
                    for r in range(8)
                ]
                for h in rh:
                    h.wait()
                grow = pl.multiple_of(
                    row_base + sid * (TILE_SLICE // L) + i * 8, 8)
                wh.append(pltpu.async_copy(buf, c_hbm.at[pl.ds(grow, 8)], wsem))
            for h in wh[-2:]:
                h.wait()
            plsc.subcore_barrier()

    return builder(src, dst)


def _attend_body(q_ref, k_ref, v_ref, c_ref, o_ref):
    lc = jnp.log(c_ref[...])
    for h in range(H):
        s = lax.dot_general(
            q_ref[h], k_ref[h], (((1,), (1,)), ((), ())),
            preferred_element_type=jnp.float32,
        )
        kf = k_ref[h].astype(jnp.float32)
        kmax = jnp.sqrt(jnp.max(jnp.sum(kf * kf, axis=1)))
        qf = q_ref[h].astype(jnp.float32)
        m = jnp.sqrt(jnp.sum(qf * qf, axis=1, keepdims=True)) * kmax
        e = jnp.exp(s - m + lc)
        denom = jnp.sum(e, axis=1, keepdims=True)
        o = lax.dot_general(
            e, v_ref[h], (((1,), (0,)), ((), ())),
            preferred_element_type=jnp.float32,
        )
        o_ref[h] = o / (denom + 1e-16)


def _attend(q, k, v, c2d, block_q=256):
    grid = (L // block_q,)
    return pl.pallas_call(
        _attend_body,
        grid=grid,
        in_specs=[
            pl.BlockSpec((H, block_q, E), lambda i: (0, i, 0)),
            pl.BlockSpec((H, L, E), lambda i: (0, 0, 0)),
            pl.BlockSpec((H, L, E), lambda i: (0, 0, 0)),
            pl.BlockSpec((block_q, L), lambda i: (i, 0)),
        ],
        out_specs=pl.BlockSpec((H, block_q, E), lambda i: (0, i, 0)),
        out_shape=jax.ShapeDtypeStruct((H, L, E), jnp.float32),
    )(q, k, v, c2d)


def kernel(queries, keys, values, adj):
    temp = 1.0 / math.sqrt(E)
    q = jnp.transpose((queries[0] * temp).astype(jnp.bfloat16), (1, 0, 2))
    k = jnp.transpose(keys[0].astype(jnp.bfloat16), (1, 0, 2))
    v = jnp.transpose(values[0].astype(jnp.bfloat16), (1, 0, 2))
    src = adj[0].astype(jnp.int32)
    dst = adj[1].astype(jnp.int32)
    c = _build_counts(src, dst)
    out = _attend(q, k, v, c)
    return jnp.transpose(out, (1, 0, 2))[None]


# speedup vs baseline: 1.1109x; 1.1109x over previous
import functools
import math

import jax
import jax.numpy as jnp
from jax import lax
from jax.experimental import pallas as pl
from jax.experimental.pallas import tpu as pltpu
from jax.experimental.pallas import tpu_sc as plsc

L = 2048
H = 16
E = 128
NNZ = 32768
NUM_SC = 2
NUM_TILES = 16
WIN_ROWS = 512
WINSZ = WIN_ROWS * L
PASSES = (L // NUM_SC) // WIN_ROWS
EDGES_PER_TILE = NNZ // NUM_TILES
TILE_SLICE = WINSZ // NUM_TILES
ZCHUNK = 8192
PAD = 256


def _build_counts(src, dst):
    mesh = plsc.VectorSubcoreMesh(core_axis_name="c", subcore_axis_name="s")

    @functools.partial(
        pl.kernel,
        out_type=jax.ShapeDtypeStruct((L, L), jnp.float32),
        mesh=mesh,
        scratch_types=[
            pltpu.VMEM((EDGES_PER_TILE,), jnp.int32),
            pltpu.VMEM((EDGES_PER_TILE,), jnp.int32),
            pltpu.VMEM((16, 128), jnp.int32),
            pltpu.VMEM((128,), jnp.float32),
            pltpu.VMEM((ZCHUNK,), jnp.float32),
            pltpu.VMEM((2, 8, L), jnp.float32),
            pltpu.VMEM_SHARED((WINSZ + PAD,), jnp.float32),
            pltpu.SemaphoreType.DMA,
            pltpu.SemaphoreType.DMA,
        ],
    )
    def builder(src_hbm, dst_hbm, c_hbm, src_v, dst_v, idx_b, ones_v, zbuf,
                bbuf, window, sem, wsem):
        cid = lax.axis_index("c")
        sid = lax.axis_index("s")

        h_src = pltpu.async_copy(
            src_hbm.at[pl.ds(sid * EDGES_PER_TILE, EDGES_PER_TILE)], src_v, sem)
        h_dst = pltpu.async_copy(
            dst_hbm.at[pl.ds(sid * EDGES_PER_TILE, EDGES_PER_TILE)], dst_v, sem)

        def fill_zeros(i, _):
            zbuf[pl.ds(i * 16, 16)] = jnp.zeros((16,), jnp.float32)
            return 0
        lax.fori_loop(0, ZCHUNK // 16, fill_zeros, 0)
        def fill_ones(i, _):
            ones_v[pl.ds(i * 16, 16)] = jnp.ones((16,), jnp.float32)
            return 0
        lax.fori_loop(0, 128 // 16, fill_ones, 0)
        h_src.wait()
        h_dst.wait()

        lane = lax.iota(jnp.int32, 16)
        dump_base = WINSZ + sid * 16

        for p in range(PASSES):
            row_base = cid * (PASSES * WIN_ROWS) + p * WIN_ROWS

            zh = [
                pltpu.async_copy(
                    zbuf, window.at[pl.ds(sid * TILE_SLICE + i * ZCHUNK, ZCHUNK)],
                    sem)
                for i in range(TILE_SLICE // ZCHUNK)
            ]
            for j in range(EDGES_PER_TILE // 128):
                for g in range(8):
                    off = j * 128 + g * 16
                    s16 = src_v[pl.ds(off, 16)]
                    d16 = dst_v[pl.ds(off, 16)]
                    r = s16 - row_base
                    in_win = (r >= 0) & (r < WIN_ROWS)
                    idx_b[j, pl.ds(g * 16, 16)] = jnp.where(
                        in_win, r * L + d16, dump_base + lane)
            for h in zh:
                h.wait()
            plsc.subcore_barrier()

            sh = [
                pltpu.async_copy(ones_v, window.at[idx_b.at[j]], sem, add=True)
                for j in range(EDGES_PER_TILE // 128)
            ]
            for h in sh:
                h.wait()
            plsc.subcore_barrier()

            wh = []
            for i in range(TILE_SLICE // (8 * L)):
                buf = bbuf.at[i % 2]
                if i >= 2:
                    wh[i - 2].wait()
                rh = [
                    pltpu.async_copy(
                        window.at[pl.ds(sid * TILE_SLICE + i * (8 * L) + r * L, L)],
                        buf.at[r], sem)
                    for r in range(8)
                ]
                for h in rh:
                    h.wait()
                grow = pl.multiple_of(
                    row_base + sid * (TILE_SLICE // L) + i * 8, 8)
                wh.append(pltpu.async_copy(buf, c_hbm.at[pl.ds(grow, 8)], wsem))
            for h in wh[-2:]:
                h.wait()
            plsc.subcore_barrier()

    return builder(src, dst)


def _attend_body(q_ref, k_ref, v_ref, c_ref, o_ref):
    lc = jnp.log(c_ref[...])
    for h in range(H):
        s = lax.dot_general(
            q_ref[h], k_ref[h], (((1,), (1,)), ((), ())),
            preferred_element_type=jnp.float32,
        )
        m = jnp.max(s, axis=1, keepdims=True)
        e = jnp.exp(s - m + lc)
        denom = jnp.sum(e, axis=1, keepdims=True)
        o = lax.dot_general(
            e, v_ref[h], (((1,), (0,)), ((), ())),
            preferred_element_type=jnp.float32,
        )
        o_ref[h] = o / (denom + 1e-16)


def _attend(q, k, v, c2d, block_q=256):
    grid = (L // block_q,)
    return pl.pallas_call(
        _attend_body,
        grid=grid,
        in_specs=[
            pl.BlockSpec((H, block_q, E), lambda i: (0, i, 0)),
            pl.BlockSpec((H, L, E), lambda i: (0, 0, 0)),
            pl.BlockSpec((H, L, E), lambda i: (0, 0, 0)),
            pl.BlockSpec((block_q, L), lambda i: (i, 0)),
        ],
        out_specs=pl.BlockSpec((H, block_q, E), lambda i: (0, i, 0)),
        out_shape=jax.ShapeDtypeStruct((H, L, E), jnp.float32),
    )(q, k, v, c2d)


def kernel(queries, keys, values, adj):
    temp = 1.0 / math.sqrt(E)
    q = jnp.transpose((queries[0] * temp).astype(jnp.bfloat16), (1, 0, 2))
    k = jnp.transpose(keys[0].astype(jnp.bfloat16), (1, 0, 2))
    v = jnp.transpose(values[0].astype(jnp.bfloat16), (1, 0, 2))
    src = adj[0].astype(jnp.int32)
    dst = adj[1].astype(jnp.int32)
    c = _build_counts(src, dst)
    out = _attend(q, k, v, c)
    return jnp.transpose(out, (1, 0, 2))[None]
